# Initial kernel scaffold; baseline (speedup 1.0000x reference)
import functools

import jax
import jax.numpy as jnp
from jax import lax
from jax.experimental import pallas as pl
from jax.experimental.pallas import tpu as pltpu
from jax.experimental.pallas import tpu_sc as plsc

N = 10000
E = 320000
NC = 2
NS = 16
EPS = E // NS
CE = 80
NCH = EPS // CE
NPS = N // NS
NDEG = 10240
W1 = 112
W2 = 48

_MESH = plsc.VectorSubcoreMesh(core_axis_name="c", subcore_axis_name="s")


def _deg_body(row_hbm, col_hbm, ew_hbm, deg_o_hbm, deg_i_hbm,
              idx_v, ew_v, hist_v, tmp_v, acc_v, shared):
    cid = lax.axis_index("c")
    sid = lax.axis_index("s")

    def zero_hist(i, _):
        hist_v[pl.ds(i * 16, 16)] = jnp.zeros((16,), jnp.float32)
        return 0
    lax.fori_loop(0, NDEG // 16, zero_hist, 0)

    def accumulate(src_hbm):
        base = sid * EPS
        def chunk(k, _):
            off = base + k * 2000
            pltpu.sync_copy(src_hbm.at[pl.ds(off, 2000)], idx_v)
            pltpu.sync_copy(ew_hbm.at[pl.ds(off, 2000)], ew_v)
            def grp(g, _):
                ii = idx_v[pl.ds(g * 16, 16)]
                ww = ew_v[pl.ds(g * 16, 16)]
                plsc.addupdate_scatter(hist_v, [ii], ww)
                return 0
            lax.fori_loop(0, 125, grp, 0)
            return 0
        lax.fori_loop(0, 10, chunk, 0)

    @pl.when(cid == 0)
    def _():
        accumulate(row_hbm)

    @pl.when(cid == 1)
    def _():
        accumulate(col_hbm)

    pltpu.sync_copy(hist_v, shared.at[sid])
    plsc.subcore_barrier()

    def zero_acc(r, _):
        acc_v[pl.ds(r * 16, 16)] = jnp.zeros((16,), jnp.float32)
        return 0
    lax.fori_loop(0, 40, zero_acc, 0)

    def reduce_one(p, _):
        pltpu.sync_copy(shared.at[p, pl.ds(sid * 640, 640)], tmp_v)
        def add16(r, _):
            sl = pl.ds(r * 16, 16)
            acc_v[sl] = acc_v[sl] + tmp_v[sl]
            return 0
        lax.fori_loop(0, 40, add16, 0)
        return 0
    lax.fori_loop(0, 16, reduce_one, 0)

    @pl.when(cid == 0)
    def _():
        pltpu.sync_copy(acc_v, deg_o_hbm.at[pl.ds(sid * 640, 640)])

    @pl.when(cid == 1)
    def _():
        pltpu.sync_copy(acc_v, deg_i_hbm.at[pl.ds(sid * 640, 640)])


_deg_call = pl.kernel(
    _deg_body,
    out_type=(jax.ShapeDtypeStruct((NDEG,), jnp.float32),
              jax.ShapeDtypeStruct((NDEG,), jnp.float32)),
    mesh=_MESH,
    scratch_types=[
        pltpu.VMEM((2000,), jnp.int32),
        pltpu.VMEM((2000,), jnp.float32),
        pltpu.VMEM((NDEG,), jnp.float32),
        pltpu.VMEM((640,), jnp.float32),
        pltpu.VMEM((640,), jnp.float32),
        pltpu.VMEM_SHARED((NS, NDEG), jnp.float32),
    ],
)


def _edge_body(width, row_hbm, col_hbm, ew_hbm, tbl_b_hbm, tbl_c_hbm,
               s_o_hbm, s_i_hbm, src_v, dst_v, ew_v, rows_v, zb_v, wb_v,
               acc_sh, sem):
    cid = lax.axis_index("c")
    sid = lax.axis_index("s")
    nf = width // 16

    def zero_zb(t, _):
        zb_v[t // nf, pl.ds((t % nf) * 16, 16)] = jnp.zeros((16,), jnp.float32)
        return 0
    lax.fori_loop(0, 125 * nf, zero_zb, 0)

    def zero_acc(j, _):
        pltpu.sync_copy(zb_v, acc_sh.at[pl.ds(sid * NPS + j * 125, 125)])
        return 0
    lax.fori_loop(0, NPS // 125, zero_acc, 0)
    plsc.subcore_barrier()

    def run_dir(src_hbm, dst_hbm, tbl_hbm):
        base = sid * EPS
        def chunk(k, _):
            off = base + k * CE
            pltpu.sync_copy(src_hbm.at[pl.ds(off, CE)], src_v)
            pltpu.sync_copy(dst_hbm.at[pl.ds(off, CE)], dst_v)
            pltpu.sync_copy(ew_hbm.at[pl.ds(off, CE)], ew_v)
            pltpu.async_copy(tbl_hbm.at[src_v], rows_v, sem).wait()
            def scale(e, _):
                bidx = jnp.full((16,), e, jnp.int32)
                wv = plsc.load_gather(ew_v, [bidx])
                for f in range(nf):
                    sl = pl.ds(f * 16, 16)
                    rows_v[e, sl] = rows_v[e, sl] * wv
                return 0
            lax.fori_loop(0, CE, scale, 0)
            pltpu.sync_copy(rows_v, acc_sh.at[dst_v], add=True)
            return 0
        lax.fori_loop(0, NCH, chunk, 0)

    @pl.when(cid == 0)
    def _():
        run_dir(row_hbm, col_hbm, tbl_b_hbm)

    @pl.when(cid == 1)
    def _():
        run_dir(col_hbm, row_hbm, tbl_c_hbm)

    plsc.subcore_barrier()

    def writeback(j, _):
        sl = pl.ds(sid * NPS + j * 125, 125)
        pltpu.sync_copy(acc_sh.at[sl], wb_v)

        @pl.when(cid == 0)
        def _():
            pltpu.sync_copy(wb_v, s_o_hbm.at[sl])

        @pl.when(cid == 1)
        def _():
            pltpu.sync_copy(wb_v, s_i_hbm.at[sl])
        return 0
    lax.fori_loop(0, NPS // 125, writeback, 0)


def _make_edge_call(width):
    return pl.kernel(
        functools.partial(_edge_body, width),
        out_type=(jax.ShapeDtypeStruct((N, width), jnp.float32),
                  jax.ShapeDtypeStruct((N, width), jnp.float32)),
        mesh=_MESH,
        scratch_types=[
            pltpu.VMEM((CE,), jnp.int32),
            pltpu.VMEM((CE,), jnp.int32),
            pltpu.VMEM((CE,), jnp.float32),
            pltpu.VMEM((CE, width), jnp.float32),
            pltpu.VMEM((125, width), jnp.float32),
            pltpu.VMEM((125, width), jnp.float32),
            pltpu.VMEM_SHARED((N, width), jnp.float32),
            pltpu.SemaphoreType.DMA,
        ],
    )


_edge_call_1 = _make_edge_call(W1)
_edge_call_2 = _make_edge_call(W2)


_BLK = 1000
_GRID = N // _BLK


def _tc1_body(x_ref, wa_ref, wb_ref, wc_ref, deg_o_ref, deg_i_ref,
              xa_ref, tb_ref, tc_ref):
    x = x_ref[...]
    r_o = 1.0 / jnp.where(deg_o_ref[...] == 0.0, 1.0, deg_o_ref[...])
    r_i = 1.0 / jnp.where(deg_i_ref[...] == 0.0, 1.0, deg_i_ref[...])
    xa_ref[...] = jnp.dot(x, wa_ref[...], preferred_element_type=jnp.float32)
    tb_ref[...] = jnp.dot(x, wb_ref[...], preferred_element_type=jnp.float32) * r_o
    tc_ref[...] = jnp.dot(x, wc_ref[...], preferred_element_type=jnp.float32) * r_i


def _tc2_body(xa_ref, so_ref, si_ref, bz_ref, bh_ref, wa_ref, wb_ref, wc_ref,
              deg_o_ref, deg_i_ref, xa2_ref, tb2_ref, tc2_ref):
    s = xa_ref[...] + so_ref[...] + si_ref[...]
    z = jax.nn.sigmoid(s[:, :50] + bz_ref[...])
    t = jnp.tanh(s[:, 50:100] + bh_ref[...])
    h1 = jax.nn.relu((1.0 - z) * t)
    r_o = 1.0 / jnp.where(deg_o_ref[...] == 0.0, 1.0, deg_o_ref[...])
    r_i = 1.0 / jnp.where(deg_i_ref[...] == 0.0, 1.0, deg_i_ref[...])
    xa2_ref[...] = jnp.dot(h1, wa_ref[...], preferred_element_type=jnp.float32)
    tb2_ref[...] = jnp.dot(h1, wb_ref[...], preferred_element_type=jnp.float32) * r_o
    tc2_ref[...] = jnp.dot(h1, wc_ref[...], preferred_element_type=jnp.float32) * r_i


def _tc3_body(xa2_ref, so2_ref, si2_ref, bz_ref, bh_ref, lw_ref, lb_ref,
              out_ref):
    s = xa2_ref[...] + so2_ref[...] + si2_ref[...]
    z = jax.nn.sigmoid(s[:, :20] + bz_ref[...])
    t = jnp.tanh(s[:, 20:40] + bh_ref[...])
    h2 = jax.nn.relu((1.0 - z) * t)
    out_ref[...] = (jnp.dot(h2, lw_ref[...], preferred_element_type=jnp.float32)
                    + lb_ref[...])


def _row_spec(w):
    return pl.BlockSpec((_BLK, w), lambda i: (i, 0))


def _full_spec(r, w):
    return pl.BlockSpec((r, w), lambda i: (0, 0))


_tc1_call = pl.pallas_call(
    _tc1_body,
    grid=(_GRID,),
    in_specs=[_row_spec(128), _full_spec(128, W1), _full_spec(128, W1),
              _full_spec(128, W1), _row_spec(1), _row_spec(1)],
    out_specs=[_row_spec(W1), _row_spec(W1), _row_spec(W1)],
    out_shape=[jax.ShapeDtypeStruct((N, W1), jnp.float32)] * 3,
)

_tc2_call = pl.pallas_call(
    _tc2_body,
    grid=(_GRID,),
    in_specs=[_row_spec(W1), _row_spec(W1), _row_spec(W1),
              _full_spec(1, 50), _full_spec(1, 50),
              _full_spec(50, W2), _full_spec(50, W2), _full_spec(50, W2),
              _row_spec(1), _row_spec(1)],
    out_specs=[_row_spec(W2), _row_spec(W2), _row_spec(W2)],
    out_shape=[jax.ShapeDtypeStruct((N, W2), jnp.float32)] * 3,
)

_tc3_call = pl.pallas_call(
    _tc3_body,
    grid=(_GRID,),
    in_specs=[_row_spec(W2), _row_spec(W2), _row_spec(W2),
              _full_spec(1, 20), _full_spec(1, 20),
              _full_spec(20, 1), _full_spec(1, 1)],
    out_specs=_row_spec(1),
    out_shape=jax.ShapeDtypeStruct((N, 1), jnp.float32),
)


def _pad_cols(w, width):
    return jnp.pad(w, ((0, 0), (0, width - w.shape[1])))


def kernel(x, edge_index, edge_weight, W1z, b1z, W1r, b1r, W1h, b1h,
           W2z, b2z, W2r, b2r, W2h, b2h, lin_W, lin_b):
    row = edge_index[0]
    col = edge_index[1]

    d1, d2 = 128, 50
    a1 = _pad_cols(jnp.concatenate(
        [W1z[0, 0, :d1] + W1z[1, 0, :d1], W1h[0, 0, :d1] + W1h[1, 0, :d1]], 1), W1)
    b1 = _pad_cols(jnp.concatenate([W1z[0, 1, :d1], W1h[0, 1, :d1]], 1), W1)
    c1 = _pad_cols(jnp.concatenate([W1z[1, 1, :d1], W1h[1, 1, :d1]], 1), W1)
    a2 = _pad_cols(jnp.concatenate(
        [W2z[0, 0, :d2] + W2z[1, 0, :d2], W2h[0, 0, :d2] + W2h[1, 0, :d2]], 1), W2)
    b2 = _pad_cols(jnp.concatenate([W2z[0, 1, :d2], W2h[0, 1, :d2]], 1), W2)
    c2 = _pad_cols(jnp.concatenate([W2z[1, 1, :d2], W2h[1, 1, :d2]], 1), W2)

    deg_o_p, deg_i_p = _deg_call(row, col, edge_weight)
    deg_o = deg_o_p[:N].reshape(N, 1)
    deg_i = deg_i_p[:N].reshape(N, 1)

    xa, tbl_b, tbl_c = _tc1_call(x, a1, b1, c1, deg_o, deg_i)
    s_o, s_i = _edge_call_1(row, col, edge_weight, tbl_b, tbl_c)
    xa2, tbl_b2, tbl_c2 = _tc2_call(xa, s_o, s_i, b1z.reshape(1, 50),
                                    b1h.reshape(1, 50), a2, b2, c2,
                                    deg_o, deg_i)
    s_o2, s_i2 = _edge_call_2(row, col, edge_weight, tbl_b2, tbl_c2)
    return _tc3_call(xa2, s_o2, s_i2, b2z.reshape(1, 20),
                     b2h.reshape(1, 20), lin_W, lin_b.reshape(1, 1))


# speedup vs baseline: 8.2099x; 8.2099x over previous
import functools

import jax
import jax.numpy as jnp
from jax import lax
from jax.experimental import pallas as pl
from jax.experimental.pallas import tpu as pltpu
from jax.experimental.pallas import tpu_sc as plsc

N = 10000
E = 320000
NC = 2
NS = 16
EPS = E // NS
CE = 80
NCH = EPS // CE
NPAD = 10240
NPS = NPAD // NS
TW = 128

_MESH = plsc.VectorSubcoreMesh(core_axis_name="c", subcore_axis_name="s")
_SC_PARAMS = pltpu.CompilerParams(needs_layout_passes=False)


def _deg_body(row_hbm, col_hbm, ew_hbm, deg_o_hbm, deg_i_hbm,
              idx_v, ew_v, hist_v, tmp_v, acc_v, shared):
    cid = lax.axis_index("c")
    sid = lax.axis_index("s")

    def zero_hist(i, _):
        hist_v[pl.ds(i * 16, 16)] = jnp.zeros((16,), jnp.float32)
        return 0
    lax.fori_loop(0, NPAD // 16, zero_hist, 0)

    def accumulate(src_hbm):
        base = sid * EPS
        def chunk(k, _):
            off = base + k * 2000
            pltpu.sync_copy(src_hbm.at[pl.ds(off, 2000)], idx_v)
            pltpu.sync_copy(ew_hbm.at[pl.ds(off, 2000)], ew_v)
            def grp(g, _):
                ii = idx_v[pl.ds(g * 16, 16)]
                ww = ew_v[pl.ds(g * 16, 16)]
                plsc.addupdate_scatter(hist_v, [ii], ww)
                return 0
            lax.fori_loop(0, 125, grp, 0)
            return 0
        lax.fori_loop(0, 10, chunk, 0)

    @pl.when(cid == 0)
    def _():
        accumulate(row_hbm)

    @pl.when(cid == 1)
    def _():
        accumulate(col_hbm)

    pltpu.sync_copy(hist_v, shared.at[sid])
    plsc.subcore_barrier()

    def zero_acc(r, _):
        acc_v[pl.ds(r * 16, 16)] = jnp.zeros((16,), jnp.float32)
        return 0
    lax.fori_loop(0, NPS // 16, zero_acc, 0)

    def reduce_one(p, _):
        pltpu.sync_copy(shared.at[p, pl.ds(sid * NPS, NPS)], tmp_v)
        def add16(r, _):
            sl = pl.ds(r * 16, 16)
            acc_v[sl] = acc_v[sl] + tmp_v[sl]
            return 0
        lax.fori_loop(0, NPS // 16, add16, 0)
        return 0
    lax.fori_loop(0, 16, reduce_one, 0)

    @pl.when(cid == 0)
    def _():
        pltpu.sync_copy(acc_v, deg_o_hbm.at[pl.ds(sid * NPS, NPS)])

    @pl.when(cid == 1)
    def _():
        pltpu.sync_copy(acc_v, deg_i_hbm.at[pl.ds(sid * NPS, NPS)])


_deg_call = pl.kernel(
    _deg_body,
    out_type=(jax.ShapeDtypeStruct((NPAD,), jnp.float32),
              jax.ShapeDtypeStruct((NPAD,), jnp.float32)),
    mesh=_MESH,
    scratch_types=[
        pltpu.VMEM((2000,), jnp.int32),
        pltpu.VMEM((2000,), jnp.float32),
        pltpu.VMEM((NPAD,), jnp.float32),
        pltpu.VMEM((NPS,), jnp.float32),
        pltpu.VMEM((NPS,), jnp.float32),
        pltpu.VMEM_SHARED((NS, NPAD), jnp.float32),
    ],
    compiler_params=_SC_PARAMS,
)


def _edge_body(ng0, off0, ng1, off1, row_hbm, col_hbm, ew_hbm,
               tbl_b_hbm, tbl_c_hbm, s_o_hbm, s_i_hbm,
               src_v, dst_v, ew_v, rows_v, zb_v, wb_v, acc_sh, sem):
    cid = lax.axis_index("c")
    sid = lax.axis_index("s")
    nfz = TW // 16

    def zero_zb(t, _):
        zb_v[t // nfz, pl.ds((t % nfz) * 16, 16)] = jnp.zeros((16,), jnp.float32)
        return 0
    lax.fori_loop(0, 128 * nfz, zero_zb, 0)

    def zero_acc(j, _):
        pltpu.sync_copy(zb_v, acc_sh.at[pl.ds(sid * NPS + j * 128, 128)])
        return 0
    lax.fori_loop(0, NPS // 128, zero_acc, 0)
    plsc.subcore_barrier()

    def run_dir(src_hbm, dst_hbm, tbl_hbm, ngroups, coff):
        base = sid * EPS
        def chunk(k, _):
            off = base + k * CE
            pltpu.sync_copy(src_hbm.at[pl.ds(off, CE)], src_v)
            pltpu.sync_copy(dst_hbm.at[pl.ds(off, CE)], dst_v)
            pltpu.sync_copy(ew_hbm.at[pl.ds(off, CE)], ew_v)
            pltpu.async_copy(tbl_hbm.at[src_v], rows_v, sem).wait()
            def scale(e, _):
                bidx = jnp.full((16,), e, jnp.int32)
                wv = plsc.load_gather(ew_v, [bidx])
                for f in range(ngroups):
                    sl = pl.ds(coff + f * 16, 16)
                    rows_v[e, sl] = rows_v[e, sl] * wv
                return 0
            lax.fori_loop(0, CE, scale, 0)
            pltpu.sync_copy(rows_v, acc_sh.at[dst_v], add=True)
            return 0
        lax.fori_loop(0, NCH, chunk, 0)

    @pl.when(cid == 0)
    def _():
        run_dir(row_hbm, col_hbm, tbl_b_hbm, ng0, off0)

    @pl.when(cid == 1)
    def _():
        run_dir(col_hbm, row_hbm, tbl_c_hbm, ng1, off1)

    plsc.subcore_barrier()

    def writeback(j, _):
        sl = pl.ds(sid * NPS + j * 128, 128)
        pltpu.sync_copy(acc_sh.at[sl], wb_v)

        @pl.when(cid == 0)
        def _():
            pltpu.sync_copy(wb_v, s_o_hbm.at[sl])

        @pl.when(cid == 1)
        def _():
            pltpu.sync_copy(wb_v, s_i_hbm.at[sl])
        return 0
    lax.fori_loop(0, NPS // 128, writeback, 0)


def _make_edge_call(ng0, off0, ng1, off1):
    return pl.kernel(
        functools.partial(_edge_body, ng0, off0, ng1, off1),
        out_type=(jax.ShapeDtypeStruct((NPAD, TW), jnp.float32),
                  jax.ShapeDtypeStruct((NPAD, TW), jnp.float32)),
        mesh=_MESH,
        scratch_types=[
            pltpu.VMEM((CE,), jnp.int32),
            pltpu.VMEM((CE,), jnp.int32),
            pltpu.VMEM((CE,), jnp.float32),
            pltpu.VMEM((CE, TW), jnp.float32),
            pltpu.VMEM((128, TW), jnp.float32),
            pltpu.VMEM((128, TW), jnp.float32),
            pltpu.VMEM_SHARED((NPAD, TW), jnp.float32),
            pltpu.SemaphoreType.DMA,
        ],
        compiler_params=_SC_PARAMS,
    )


_edge_call_1 = _make_edge_call(7, 0, 7, 0)
_edge_call_2 = _make_edge_call(3, 0, 3, 64)


_BLK = 1000
_GRID = N // _BLK


def _tc1_body(x_ref, wa_ref, wb_ref, wc_ref, deg_o_ref, deg_i_ref,
              xa_ref, tb_ref, tc_ref):
    x = x_ref[...]
    r_o = 1.0 / jnp.where(deg_o_ref[...] == 0.0, 1.0, deg_o_ref[...])
    r_i = 1.0 / jnp.where(deg_i_ref[...] == 0.0, 1.0, deg_i_ref[...])
    xa_ref[...] = jnp.dot(x, wa_ref[...], preferred_element_type=jnp.float32)
    tb_ref[...] = jnp.dot(x, wb_ref[...], preferred_element_type=jnp.float32) * r_o
    tc_ref[...] = jnp.dot(x, wc_ref[...], preferred_element_type=jnp.float32) * r_i


def _tc2_body(xa_ref, so_ref, si_ref, bz_ref, bh_ref, wa2_ref, wbc_ref,
              deg_o_ref, deg_i_ref, xa2_ref, tbl2_ref):
    s = xa_ref[...] + so_ref[...] + si_ref[...]
    z = jax.nn.sigmoid(s[:, :50] + bz_ref[...])
    t = jnp.tanh(s[:, 50:100] + bh_ref[...])
    h1 = jax.nn.relu((1.0 - z) * t)
    r_o = 1.0 / jnp.where(deg_o_ref[...] == 0.0, 1.0, deg_o_ref[...])
    r_i = 1.0 / jnp.where(deg_i_ref[...] == 0.0, 1.0, deg_i_ref[...])
    y = jnp.dot(h1, wbc_ref[...], preferred_element_type=jnp.float32)
    scale = jnp.concatenate([jnp.broadcast_to(r_o, (_BLK, 64)),
                             jnp.broadcast_to(r_i, (_BLK, 64))], axis=1)
    tbl2_ref[...] = y * scale
    xa2_ref[...] = jnp.dot(h1, wa2_ref[...], preferred_element_type=jnp.float32)


def _tc3_body(xa2_ref, so2_ref, si2_ref, bz_ref, bh_ref, lw_ref, lb_ref,
              out_ref):
    s_z = xa2_ref[:, :20] + so2_ref[:, :20] + si2_ref[:, 64:84] + bz_ref[...]
    s_h = xa2_ref[:, 20:40] + so2_ref[:, 20:40] + si2_ref[:, 84:104] + bh_ref[...]
    z = jax.nn.sigmoid(s_z)
    t = jnp.tanh(s_h)
    h2 = jax.nn.relu((1.0 - z) * t)
    out_ref[...] = (jnp.dot(h2, lw_ref[...], preferred_element_type=jnp.float32)
                    + lb_ref[...])


def _row_spec(w):
    return pl.BlockSpec((_BLK, w), lambda i: (i, 0))


def _full_spec(r, w):
    return pl.BlockSpec((r, w), lambda i: (0, 0))


_tc1_call = pl.pallas_call(
    _tc1_body,
    grid=(_GRID,),
    in_specs=[_row_spec(128), _full_spec(128, TW), _full_spec(128, TW),
              _full_spec(128, TW), _row_spec(1), _row_spec(1)],
    out_specs=[_row_spec(TW), _row_spec(TW), _row_spec(TW)],
    out_shape=[jax.ShapeDtypeStruct((N, TW), jnp.float32)] * 3,
)

_tc2_call = pl.pallas_call(
    _tc2_body,
    grid=(_GRID,),
    in_specs=[_row_spec(TW), _row_spec(TW), _row_spec(TW),
              _full_spec(1, 50), _full_spec(1, 50),
              _full_spec(50, 48), _full_spec(50, TW),
              _row_spec(1), _row_spec(1)],
    out_specs=[_row_spec(48), _row_spec(TW)],
    out_shape=[jax.ShapeDtypeStruct((N, 48), jnp.float32),
               jax.ShapeDtypeStruct((N, TW), jnp.float32)],
)

_tc3_call = pl.pallas_call(
    _tc3_body,
    grid=(_GRID,),
    in_specs=[_row_spec(48), _row_spec(TW), _row_spec(TW),
              _full_spec(1, 20), _full_spec(1, 20),
              _full_spec(20, 1), _full_spec(1, 1)],
    out_specs=_row_spec(1),
    out_shape=jax.ShapeDtypeStruct((N, 1), jnp.float32),
)


def _pad_cols(w, width):
    return jnp.pad(w, ((0, 0), (0, width - w.shape[1])))


def kernel(x, edge_index, edge_weight, W1z, b1z, W1r, b1r, W1h, b1h,
           W2z, b2z, W2r, b2r, W2h, b2h, lin_W, lin_b):
    row = edge_index[0]
    col = edge_index[1]

    d1, d2 = 128, 50
    a1 = _pad_cols(jnp.concatenate(
        [W1z[0, 0, :d1] + W1z[1, 0, :d1], W1h[0, 0, :d1] + W1h[1, 0, :d1]], 1), TW)
    b1 = _pad_cols(jnp.concatenate([W1z[0, 1, :d1], W1h[0, 1, :d1]], 1), TW)
    c1 = _pad_cols(jnp.concatenate([W1z[1, 1, :d1], W1h[1, 1, :d1]], 1), TW)
    a2 = _pad_cols(jnp.concatenate(
        [W2z[0, 0, :d2] + W2z[1, 0, :d2], W2h[0, 0, :d2] + W2h[1, 0, :d2]], 1), 48)
    w2bc = jnp.concatenate(
        [_pad_cols(jnp.concatenate([W2z[0, 1, :d2], W2h[0, 1, :d2]], 1), 64),
         _pad_cols(jnp.concatenate([W2z[1, 1, :d2], W2h[1, 1, :d2]], 1), 64)], 1)

    deg_o_p, deg_i_p = _deg_call(row, col, edge_weight)
    deg_o = deg_o_p[:N].reshape(N, 1)
    deg_i = deg_i_p[:N].reshape(N, 1)

    xa, tbl_b, tbl_c = _tc1_call(x, a1, b1, c1, deg_o, deg_i)
    s_o, s_i = _edge_call_1(row, col, edge_weight, tbl_b, tbl_c)
    xa2, tbl2 = _tc2_call(xa, s_o[:N], s_i[:N], b1z.reshape(1, 50),
                          b1h.reshape(1, 50), a2, w2bc, deg_o, deg_i)
    s_o2, s_i2 = _edge_call_2(row, col, edge_weight, tbl2, tbl2)
    return _tc3_call(xa2, s_o2[:N], s_i2[:N], b2z.reshape(1, 20),
                     b2h.reshape(1, 20), lin_W, lin_b.reshape(1, 1))
